# Initial kernel scaffold; baseline (speedup 1.0000x reference)
import jax
import jax.numpy as jnp
from jax.experimental import pallas as pl


def kernel(x, edge_index, W1l, b1l, W1r, g1, be1, W2l, b2l, W2r, g2, be2, Wm1, bm1, Wm2, bm2):
    raise NotImplementedError("write your pallas kernel here")


# speedup vs baseline: 27.9480x; 27.9480x over previous
import functools

import jax
import jax.numpy as jnp
from jax import lax
from jax.experimental import pallas as pl
from jax.experimental.pallas import tpu as pltpu
from jax.experimental.pallas import tpu_sc as plsc

FP = 16
LANES = 128
KDEPTH = 8
NSC = 2
NTILE = 16
BLK = 2000


def _sc_mesh():
    return plsc.VectorSubcoreMesh(
        core_axis_name="c", subcore_axis_name="s", num_cores=NSC,
        num_subcores=NTILE)


def _make_sc_agg1(np_, r_pad):
    stripe = np_ // NTILE
    rows_per_tile = r_pad // (NSC * NTILE)
    nslab = rows_per_tile // KDEPTH

    def body(tab, srcr, dstr, zrows, out_a, out_b, srcb, dstb, rowsb, acc,
             sem):
        c = lax.axis_index("c")
        s = lax.axis_index("s")
        pltpu.sync_copy(zrows, acc.at[pl.ds(s * stripe, stripe)])
        plsc.subcore_barrier()
        lo = (c * NTILE + s) * rows_per_tile

        def slab(t, carry):
            r0 = lo + t * KDEPTH
            pltpu.sync_copy(srcr.at[pl.ds(r0, KDEPTH)], srcb)
            pltpu.sync_copy(dstr.at[pl.ds(r0, KDEPTH)], dstb)
            descs = [
                pltpu.async_copy(tab.at[srcb.at[j]], rowsb.at[j], sem)
                for j in range(KDEPTH)
            ]
            for d in descs:
                d.wait()
            for j in range(KDEPTH):
                pltpu.sync_copy(rowsb.at[j], acc.at[dstb.at[j]], add=True)
            return carry

        lax.fori_loop(0, nslab, slab, 0)
        plsc.subcore_barrier()

        @pl.when(c == 0)
        def _():
            pltpu.sync_copy(acc.at[pl.ds(s * stripe, stripe)],
                            out_a.at[pl.ds(s * stripe, stripe)])

        @pl.when(c == 1)
        def _():
            pltpu.sync_copy(acc.at[pl.ds(s * stripe, stripe)],
                            out_b.at[pl.ds(s * stripe, stripe)])

    return pl.kernel(
        body,
        out_type=(jax.ShapeDtypeStruct((np_, FP), jnp.float32),
                  jax.ShapeDtypeStruct((np_, FP), jnp.float32)),
        mesh=_sc_mesh(),
        compiler_params=pltpu.CompilerParams(use_tc_tiling_on_sc=False),
        scratch_types=[
            pltpu.VMEM((KDEPTH, LANES), jnp.int32),
            pltpu.VMEM((KDEPTH, LANES), jnp.int32),
            pltpu.VMEM((KDEPTH, LANES, FP), jnp.float32),
            pltpu.VMEM_SHARED((np_, FP), jnp.float32),
            pltpu.SemaphoreType.DMA,
        ],
    )


def _make_sc_agg2(np_, r_pad):
    stripe = np_ // NTILE
    rows_per_tile = r_pad // NTILE
    nslab = rows_per_tile // KDEPTH

    def body(tab_a, tab_b, srcr, dstr, zrows, out_a, out_b, srcb, dstb,
             rowsb, acc, sem):
        c = lax.axis_index("c")
        s = lax.axis_index("s")
        pltpu.sync_copy(zrows, acc.at[pl.ds(s * stripe, stripe)])
        plsc.subcore_barrier()
        lo = s * rows_per_tile

        def run(tab):
            def slab(t, carry):
                r0 = lo + t * KDEPTH
                pltpu.sync_copy(srcr.at[pl.ds(r0, KDEPTH)], srcb)
                pltpu.sync_copy(dstr.at[pl.ds(r0, KDEPTH)], dstb)
                descs = [
                    pltpu.async_copy(tab.at[srcb.at[j]], rowsb.at[j], sem)
                    for j in range(KDEPTH)
                ]
                for d in descs:
                    d.wait()
                for j in range(KDEPTH):
                    pltpu.sync_copy(rowsb.at[j], acc.at[dstb.at[j]],
                                    add=True)
                return carry

            lax.fori_loop(0, nslab, slab, 0)

        @pl.when(c == 0)
        def _():
            run(tab_a)

        @pl.when(c == 1)
        def _():
            run(tab_b)

        plsc.subcore_barrier()

        @pl.when(c == 0)
        def _():
            pltpu.sync_copy(acc.at[pl.ds(s * stripe, stripe)],
                            out_a.at[pl.ds(s * stripe, stripe)])

        @pl.when(c == 1)
        def _():
            pltpu.sync_copy(acc.at[pl.ds(s * stripe, stripe)],
                            out_b.at[pl.ds(s * stripe, stripe)])

    return pl.kernel(
        body,
        out_type=(jax.ShapeDtypeStruct((np_, FP), jnp.float32),
                  jax.ShapeDtypeStruct((np_, FP), jnp.float32)),
        mesh=_sc_mesh(),
        compiler_params=pltpu.CompilerParams(use_tc_tiling_on_sc=False),
        scratch_types=[
            pltpu.VMEM((KDEPTH, LANES), jnp.int32),
            pltpu.VMEM((KDEPTH, LANES), jnp.int32),
            pltpu.VMEM((KDEPTH, LANES, FP), jnp.float32),
            pltpu.VMEM_SHARED((np_, FP), jnp.float32),
            pltpu.SemaphoreType.DMA,
        ],
    )


def _tc1a_body(a0, a1, xp, wl, wr, b, t_ref, st_ref, inv_ref, ssum, ssq):
    i = pl.program_id(0)
    cnt = a0[:, FP - 1:FP] + a1[:, FP - 1:FP]
    inv = 1.0 / jnp.maximum(cnt, 1.0)
    agg = (a0[...] + a1[...]) * inv
    t = (jnp.dot(agg, wl[...], preferred_element_type=jnp.float32)
         + jnp.dot(xp[...], wr[...], preferred_element_type=jnp.float32)
         + b[...])
    t_ref[...] = t
    inv_ref[...] = inv

    @pl.when(i == 0)
    def _():
        ssum[...] = jnp.zeros_like(ssum)
        ssq[...] = jnp.zeros_like(ssq)

    ssum[...] += jnp.sum(t, axis=0, keepdims=True)
    ssq[...] += jnp.sum(t * t, axis=0, keepdims=True)
    st_ref[0:1, :] = ssum[...]
    st_ref[1:2, :] = ssq[...]


def _tc1b_body(n, t_in, st, g, be, ha, hb):
    rn = 1.0 / n
    mean = st[0:1, :] * rn
    var = st[1:2, :] * rn - mean * mean
    isd = lax.rsqrt(var + 1e-5)
    h = jnp.maximum((t_in[...] - mean) * isd * g[...] + be[...], 0.0)
    ha[...] = h[:, :FP]
    hb[...] = h[:, FP:]


def _tc2a_body(a2a, a2b, h1a, h1b, invc, wl, wr, b, t_ref, st_ref, ssum,
               ssq):
    i = pl.program_id(0)
    inv = invc[...]
    t = (jnp.dot(a2a[...] * inv, wl[0:FP, :],
                 preferred_element_type=jnp.float32)
         + jnp.dot(a2b[...] * inv, wl[FP:2 * FP, :],
                   preferred_element_type=jnp.float32)
         + jnp.dot(h1a[...], wr[0:FP, :],
                   preferred_element_type=jnp.float32)
         + jnp.dot(h1b[...], wr[FP:2 * FP, :],
                   preferred_element_type=jnp.float32)
         + b[...])
    t_ref[...] = t

    @pl.when(i == 0)
    def _():
        ssum[...] = jnp.zeros_like(ssum)
        ssq[...] = jnp.zeros_like(ssq)

    ssum[...] += jnp.sum(t, axis=0, keepdims=True)
    ssq[...] += jnp.sum(t * t, axis=0, keepdims=True)
    st_ref[0:1, :] = ssum[...]
    st_ref[1:2, :] = ssq[...]


def _tc2b_body(n, t_in, st, g, be, wm1, bm1, wm2, bm2, out):
    rn = 1.0 / n
    mean = st[0:1, :] * rn
    var = st[1:2, :] * rn - mean * mean
    isd = lax.rsqrt(var + 1e-5)
    h = jnp.maximum((t_in[...] - mean) * isd * g[...] + be[...], 0.0)
    m = jnp.maximum(
        jnp.dot(h, wm1[...], preferred_element_type=jnp.float32) + bm1[...],
        0.0)
    out[...] = (jnp.dot(m, wm2[...], preferred_element_type=jnp.float32)
                + bm2[...])


def kernel(x, edge_index, W1l, b1l, W1r, g1, be1, W2l, b2l, W2r, g2, be2,
           Wm1, bm1, Wm2, bm2):
    n, f = x.shape
    e = edge_index.shape[1]
    nb = n // BLK
    np_ = (n // 128 + 1) * 128

    src = edge_index[0].astype(jnp.int32)
    dst = edge_index[1].astype(jnp.int32)
    unit = NSC * NTILE * KDEPTH * LANES
    e_pad = ((e + unit - 1) // unit) * unit
    pad = e_pad - e
    if pad:
        src = jnp.concatenate([src, jnp.zeros((pad,), jnp.int32)])
        dst = jnp.concatenate([dst, jnp.full((pad,), n, jnp.int32)])
    srcr = src.reshape(-1, LANES)
    dstr = dst.reshape(-1, LANES)
    r_pad = e_pad // LANES

    xp = jnp.concatenate(
        [x, jnp.zeros((n, FP - 1 - f), x.dtype), jnp.ones((n, 1), x.dtype)],
        axis=1)
    zrows = jnp.zeros((np_ // NTILE, FP), jnp.float32)
    w1l = jnp.concatenate([W1l, jnp.zeros((FP - f, W1l.shape[1]))], axis=0)
    w1r = jnp.concatenate([W1r, jnp.zeros((FP - f, W1r.shape[1]))], axis=0)

    a1a, a1b = _make_sc_agg1(np_, r_pad)(xp, srcr, dstr, zrows)

    row_spec = pl.BlockSpec((BLK, FP), lambda i: (i, 0))
    full = lambda shape: pl.BlockSpec(shape, lambda i: (0, 0))

    t1, st1, invc = pl.pallas_call(
        _tc1a_body,
        grid=(nb,),
        in_specs=[row_spec, row_spec, row_spec, full((FP, 32)),
                  full((FP, 32)), full((1, 32))],
        out_specs=[pl.BlockSpec((BLK, 32), lambda i: (i, 0)),
                   full((2, 32)),
                   pl.BlockSpec((BLK, 1), lambda i: (i, 0))],
        out_shape=[jax.ShapeDtypeStruct((n, 32), jnp.float32),
                   jax.ShapeDtypeStruct((2, 32), jnp.float32),
                   jax.ShapeDtypeStruct((n, 1), jnp.float32)],
        scratch_shapes=[pltpu.VMEM((1, 32), jnp.float32),
                        pltpu.VMEM((1, 32), jnp.float32)],
    )(a1a, a1b, xp, w1l, w1r, b1l.reshape(1, -1))

    h1a, h1b = pl.pallas_call(
        functools.partial(_tc1b_body, n),
        grid=(nb,),
        in_specs=[pl.BlockSpec((BLK, 32), lambda i: (i, 0)), full((2, 32)),
                  full((1, 32)), full((1, 32))],
        out_specs=[row_spec, row_spec],
        out_shape=[jax.ShapeDtypeStruct((n, FP), jnp.float32),
                   jax.ShapeDtypeStruct((n, FP), jnp.float32)],
    )(t1, st1, g1.reshape(1, -1), be1.reshape(1, -1))

    a2a, a2b = _make_sc_agg2(np_, r_pad)(h1a, h1b, srcr, dstr, zrows)

    t2, st2 = pl.pallas_call(
        _tc2a_body,
        grid=(nb,),
        in_specs=[row_spec, row_spec, row_spec, row_spec,
                  pl.BlockSpec((BLK, 1), lambda i: (i, 0)),
                  full((32, 32)), full((32, 32)), full((1, 32))],
        out_specs=[pl.BlockSpec((BLK, 32), lambda i: (i, 0)),
                   full((2, 32))],
        out_shape=[jax.ShapeDtypeStruct((n, 32), jnp.float32),
                   jax.ShapeDtypeStruct((2, 32), jnp.float32)],
        scratch_shapes=[pltpu.VMEM((1, 32), jnp.float32),
                        pltpu.VMEM((1, 32), jnp.float32)],
    )(a2a, a2b, h1a, h1b, invc, W2l, W2r, b2l.reshape(1, -1))

    out = pl.pallas_call(
        functools.partial(_tc2b_body, n),
        grid=(nb,),
        in_specs=[pl.BlockSpec((BLK, 32), lambda i: (i, 0)), full((2, 32)),
                  full((1, 32)), full((1, 32)), full((32, 64)),
                  full((1, 64)), full((64, 1)), full((1, 1))],
        out_specs=pl.BlockSpec((BLK, 1), lambda i: (i, 0)),
        out_shape=jax.ShapeDtypeStruct((n, 1), jnp.float32),
    )(t2, st2, g2.reshape(1, -1), be2.reshape(1, -1), Wm1,
      bm1.reshape(1, -1), Wm2, bm2.reshape(1, -1))

    return out.reshape(n)


# speedup vs baseline: 35.5572x; 1.2723x over previous
import functools

import jax
import jax.numpy as jnp
from jax import lax
from jax.experimental import pallas as pl
from jax.experimental.pallas import tpu as pltpu
from jax.experimental.pallas import tpu_sc as plsc

FP = 16
LANES = 128
KDEPTH = 6
NSC = 2
NTILE = 16
BLK = 2000


def _sc_mesh():
    return plsc.VectorSubcoreMesh(
        core_axis_name="c", subcore_axis_name="s", num_cores=NSC,
        num_subcores=NTILE)


def _agg_loop(tab, edg, idxb, rowsb, acc, gsem, lo, nslab):
    def fetch(b, t):
        r0 = lo + t * KDEPTH
        pltpu.sync_copy(edg.at[pl.ds(r0, KDEPTH)], idxb.at[b])
        for j in range(KDEPTH):
            pltpu.async_copy(tab.at[idxb.at[b, j, 0]], rowsb.at[b, j], gsem)

    fetch(0, 0)

    def slab(t, carry):
        b = lax.rem(t, 2)
        for j in range(KDEPTH):
            pltpu.make_async_copy(tab.at[idxb.at[b, j, 0]], rowsb.at[b, j],
                                  gsem).wait()

        @pl.when(t + 1 < nslab)
        def _():
            fetch(1 - b, t + 1)

        for j in range(KDEPTH):
            pltpu.sync_copy(rowsb.at[b, j], acc.at[idxb.at[b, j, 1]],
                            add=True)
        return carry

    lax.fori_loop(0, nslab, slab, 0)


def _sc_scratch(np_):
    return [
        pltpu.VMEM((2, KDEPTH, 2, LANES), jnp.int32),
        pltpu.VMEM((2, KDEPTH, LANES, FP), jnp.float32),
        pltpu.VMEM_SHARED((np_, FP), jnp.float32),
        pltpu.SemaphoreType.DMA,
    ]


def _make_sc_agg1(np_, r_pad):
    stripe = np_ // NTILE
    rows_per_tile = r_pad // (NSC * NTILE)
    nslab = rows_per_tile // KDEPTH

    def body(tab, edg, zrows, out_a, out_b, idxb, rowsb, acc, sem):
        c = lax.axis_index("c")
        s = lax.axis_index("s")
        pltpu.sync_copy(zrows, acc.at[pl.ds(s * stripe, stripe)])
        plsc.subcore_barrier()
        lo = (c * NTILE + s) * rows_per_tile
        _agg_loop(tab, edg, idxb, rowsb, acc, sem, lo, nslab)
        plsc.subcore_barrier()

        @pl.when(c == 0)
        def _():
            pltpu.sync_copy(acc.at[pl.ds(s * stripe, stripe)],
                            out_a.at[pl.ds(s * stripe, stripe)])

        @pl.when(c == 1)
        def _():
            pltpu.sync_copy(acc.at[pl.ds(s * stripe, stripe)],
                            out_b.at[pl.ds(s * stripe, stripe)])

    return pl.kernel(
        body,
        out_type=(jax.ShapeDtypeStruct((np_, FP), jnp.float32),
                  jax.ShapeDtypeStruct((np_, FP), jnp.float32)),
        mesh=_sc_mesh(),
        compiler_params=pltpu.CompilerParams(use_tc_tiling_on_sc=False),
        scratch_types=_sc_scratch(np_),
    )


def _make_sc_agg2(np_, r_pad):
    stripe = np_ // NTILE
    rows_per_tile = r_pad // NTILE
    nslab = rows_per_tile // KDEPTH

    def body(tab_a, tab_b, edg, zrows, out_a, out_b, idxb, rowsb, acc, sem):
        c = lax.axis_index("c")
        s = lax.axis_index("s")
        pltpu.sync_copy(zrows, acc.at[pl.ds(s * stripe, stripe)])
        plsc.subcore_barrier()
        lo = s * rows_per_tile

        @pl.when(c == 0)
        def _():
            _agg_loop(tab_a, edg, idxb, rowsb, acc, sem, lo, nslab)

        @pl.when(c == 1)
        def _():
            _agg_loop(tab_b, edg, idxb, rowsb, acc, sem, lo, nslab)

        plsc.subcore_barrier()

        @pl.when(c == 0)
        def _():
            pltpu.sync_copy(acc.at[pl.ds(s * stripe, stripe)],
                            out_a.at[pl.ds(s * stripe, stripe)])

        @pl.when(c == 1)
        def _():
            pltpu.sync_copy(acc.at[pl.ds(s * stripe, stripe)],
                            out_b.at[pl.ds(s * stripe, stripe)])

    return pl.kernel(
        body,
        out_type=(jax.ShapeDtypeStruct((np_, FP), jnp.float32),
                  jax.ShapeDtypeStruct((np_, FP), jnp.float32)),
        mesh=_sc_mesh(),
        compiler_params=pltpu.CompilerParams(use_tc_tiling_on_sc=False),
        scratch_types=_sc_scratch(np_),
    )


def _tc1a_body(a0, a1, xp, wl, wr, b, t_ref, st_ref, inv_ref, ssum, ssq):
    i = pl.program_id(0)
    cnt = a0[:, FP - 1:FP] + a1[:, FP - 1:FP]
    inv = 1.0 / jnp.maximum(cnt, 1.0)
    agg = (a0[...] + a1[...]) * inv
    t = (jnp.dot(agg, wl[...], preferred_element_type=jnp.float32)
         + jnp.dot(xp[...], wr[...], preferred_element_type=jnp.float32)
         + b[...])
    t_ref[...] = t
    inv_ref[...] = inv

    @pl.when(i == 0)
    def _():
        ssum[...] = jnp.zeros_like(ssum)
        ssq[...] = jnp.zeros_like(ssq)

    ssum[...] += jnp.sum(t, axis=0, keepdims=True)
    ssq[...] += jnp.sum(t * t, axis=0, keepdims=True)
    st_ref[0:1, :] = ssum[...]
    st_ref[1:2, :] = ssq[...]


def _tc1b_body(n, t_in, st, g, be, ha, hb):
    rn = 1.0 / n
    mean = st[0:1, :] * rn
    var = st[1:2, :] * rn - mean * mean
    isd = lax.rsqrt(var + 1e-5)
    h = jnp.maximum((t_in[...] - mean) * isd * g[...] + be[...], 0.0)
    ha[...] = h[:, :FP]
    hb[...] = h[:, FP:]


def _tc2a_body(a2a, a2b, h1a, h1b, invc, wl, wr, b, t_ref, st_ref, ssum,
               ssq):
    i = pl.program_id(0)
    inv = invc[...]
    t = (jnp.dot(a2a[...] * inv, wl[0:FP, :],
                 preferred_element_type=jnp.float32)
         + jnp.dot(a2b[...] * inv, wl[FP:2 * FP, :],
                   preferred_element_type=jnp.float32)
         + jnp.dot(h1a[...], wr[0:FP, :],
                   preferred_element_type=jnp.float32)
         + jnp.dot(h1b[...], wr[FP:2 * FP, :],
                   preferred_element_type=jnp.float32)
         + b[...])
    t_ref[...] = t

    @pl.when(i == 0)
    def _():
        ssum[...] = jnp.zeros_like(ssum)
        ssq[...] = jnp.zeros_like(ssq)

    ssum[...] += jnp.sum(t, axis=0, keepdims=True)
    ssq[...] += jnp.sum(t * t, axis=0, keepdims=True)
    st_ref[0:1, :] = ssum[...]
    st_ref[1:2, :] = ssq[...]


def _tc2b_body(n, t_in, st, g, be, wm1, bm1, wm2, bm2, out):
    rn = 1.0 / n
    mean = st[0:1, :] * rn
    var = st[1:2, :] * rn - mean * mean
    isd = lax.rsqrt(var + 1e-5)
    h = jnp.maximum((t_in[...] - mean) * isd * g[...] + be[...], 0.0)
    m = jnp.maximum(
        jnp.dot(h, wm1[...], preferred_element_type=jnp.float32) + bm1[...],
        0.0)
    out[...] = (jnp.dot(m, wm2[...], preferred_element_type=jnp.float32)
                + bm2[...])


def kernel(x, edge_index, W1l, b1l, W1r, g1, be1, W2l, b2l, W2r, g2, be2,
           Wm1, bm1, Wm2, bm2):
    n, f = x.shape
    e = edge_index.shape[1]
    nb = n // BLK
    np_ = (n // 128 + 1) * 128

    src = edge_index[0].astype(jnp.int32)
    dst = edge_index[1].astype(jnp.int32)
    unit = NSC * NTILE * KDEPTH * LANES
    e_pad = ((e + unit - 1) // unit) * unit
    pad = e_pad - e
    if pad:
        src = jnp.concatenate([src, jnp.zeros((pad,), jnp.int32)])
        dst = jnp.concatenate([dst, jnp.full((pad,), n, jnp.int32)])
    edg = jnp.stack([src.reshape(-1, LANES), dst.reshape(-1, LANES)], axis=1)
    r_pad = e_pad // LANES

    xp = jnp.concatenate(
        [x, jnp.zeros((n, FP - 1 - f), x.dtype), jnp.ones((n, 1), x.dtype)],
        axis=1)
    zrows = jnp.zeros((np_ // NTILE, FP), jnp.float32)
    w1l = jnp.concatenate([W1l, jnp.zeros((FP - f, W1l.shape[1]))], axis=0)
    w1r = jnp.concatenate([W1r, jnp.zeros((FP - f, W1r.shape[1]))], axis=0)

    a1a, a1b = _make_sc_agg1(np_, r_pad)(xp, edg, zrows)

    row_spec = pl.BlockSpec((BLK, FP), lambda i: (i, 0))
    full = lambda shape: pl.BlockSpec(shape, lambda i: (0, 0))

    t1, st1, invc = pl.pallas_call(
        _tc1a_body,
        grid=(nb,),
        in_specs=[row_spec, row_spec, row_spec, full((FP, 32)),
                  full((FP, 32)), full((1, 32))],
        out_specs=[pl.BlockSpec((BLK, 32), lambda i: (i, 0)),
                   full((2, 32)),
                   pl.BlockSpec((BLK, 1), lambda i: (i, 0))],
        out_shape=[jax.ShapeDtypeStruct((n, 32), jnp.float32),
                   jax.ShapeDtypeStruct((2, 32), jnp.float32),
                   jax.ShapeDtypeStruct((n, 1), jnp.float32)],
        scratch_shapes=[pltpu.VMEM((1, 32), jnp.float32),
                        pltpu.VMEM((1, 32), jnp.float32)],
    )(a1a, a1b, xp, w1l, w1r, b1l.reshape(1, -1))

    h1a, h1b = pl.pallas_call(
        functools.partial(_tc1b_body, n),
        grid=(nb,),
        in_specs=[pl.BlockSpec((BLK, 32), lambda i: (i, 0)), full((2, 32)),
                  full((1, 32)), full((1, 32))],
        out_specs=[row_spec, row_spec],
        out_shape=[jax.ShapeDtypeStruct((n, FP), jnp.float32),
                   jax.ShapeDtypeStruct((n, FP), jnp.float32)],
    )(t1, st1, g1.reshape(1, -1), be1.reshape(1, -1))

    a2a, a2b = _make_sc_agg2(np_, r_pad)(h1a, h1b, edg, zrows)

    t2, st2 = pl.pallas_call(
        _tc2a_body,
        grid=(nb,),
        in_specs=[row_spec, row_spec, row_spec, row_spec,
                  pl.BlockSpec((BLK, 1), lambda i: (i, 0)),
                  full((32, 32)), full((32, 32)), full((1, 32))],
        out_specs=[pl.BlockSpec((BLK, 32), lambda i: (i, 0)),
                   full((2, 32))],
        out_shape=[jax.ShapeDtypeStruct((n, 32), jnp.float32),
                   jax.ShapeDtypeStruct((2, 32), jnp.float32)],
        scratch_shapes=[pltpu.VMEM((1, 32), jnp.float32),
                        pltpu.VMEM((1, 32), jnp.float32)],
    )(a2a, a2b, h1a, h1b, invc, W2l, W2r, b2l.reshape(1, -1))

    out = pl.pallas_call(
        functools.partial(_tc2b_body, n),
        grid=(nb,),
        in_specs=[pl.BlockSpec((BLK, 32), lambda i: (i, 0)), full((2, 32)),
                  full((1, 32)), full((1, 32)), full((32, 64)),
                  full((1, 64)), full((64, 1)), full((1, 1))],
        out_specs=pl.BlockSpec((BLK, 1), lambda i: (i, 0)),
        out_shape=jax.ShapeDtypeStruct((n, 1), jnp.float32),
    )(t2, st2, g2.reshape(1, -1), be2.reshape(1, -1), Wm1,
      bm1.reshape(1, -1), Wm2, bm2.reshape(1, -1))

    return out.reshape(n)


# speedup vs baseline: 40.9039x; 1.1504x over previous
import functools

import jax
import jax.numpy as jnp
from jax import lax
from jax.experimental import pallas as pl
from jax.experimental.pallas import tpu as pltpu
from jax.experimental.pallas import tpu_sc as plsc

FP = 16
LANES = 128
KDEPTH = 5
NSC = 2
NTILE = 16
BLK = 2000


def _sc_mesh():
    return plsc.VectorSubcoreMesh(
        core_axis_name="c", subcore_axis_name="s", num_cores=NSC,
        num_subcores=NTILE)


def _agg_loop(tab, edg, idxb, rowsb, acc, gsem, ssem, isem, lo, nslab):
    def idx_load(t):
        pltpu.async_copy(edg.at[pl.ds(lo + t * KDEPTH, KDEPTH)],
                         idxb.at[lax.rem(t, 3)], isem)

    def gathers(t):
        p = lax.rem(t, 3)
        b = lax.rem(t, 2)
        for j in range(KDEPTH):
            pltpu.async_copy(tab.at[idxb.at[p, j, 0]], rowsb.at[b, j], gsem)

    def drain_scatters():
        for j in range(KDEPTH):
            pltpu.make_async_copy(rowsb.at[0, j], acc.at[idxb.at[0, j, 1]],
                                  ssem).wait()

    pltpu.sync_copy(edg.at[pl.ds(lo, KDEPTH)], idxb.at[0])
    gathers(0)
    if nslab > 1:
        idx_load(1)

    def slab(t, carry):
        p = lax.rem(t, 3)
        b = lax.rem(t, 2)
        for j in range(KDEPTH):
            pltpu.make_async_copy(tab.at[idxb.at[p, j, 0]], rowsb.at[b, j],
                                  gsem).wait()
        for j in range(KDEPTH):
            pltpu.async_copy(rowsb.at[b, j], acc.at[idxb.at[p, j, 1]], ssem,
                             add=True)

        @pl.when(t + 1 < nslab)
        def _():
            pltpu.make_async_copy(edg.at[pl.ds(lo, KDEPTH)], idxb.at[0],
                                  isem).wait()

        @pl.when(t >= 1)
        def _():
            drain_scatters()

        @pl.when(t + 2 < nslab)
        def _():
            idx_load(t + 2)

        @pl.when(t + 1 < nslab)
        def _():
            gathers(t + 1)

        return carry

    lax.fori_loop(0, nslab, slab, 0)
    drain_scatters()


def _sc_scratch(np_):
    return [
        pltpu.VMEM((3, KDEPTH, 2, LANES), jnp.int32),
        pltpu.VMEM((2, KDEPTH, LANES, FP), jnp.float32),
        pltpu.VMEM_SHARED((np_, FP), jnp.float32),
        pltpu.SemaphoreType.DMA,
        pltpu.SemaphoreType.DMA,
        pltpu.SemaphoreType.DMA,
    ]


def _make_sc_agg1(np_, r_pad):
    stripe = np_ // NTILE
    rows_per_tile = r_pad // (NSC * NTILE)
    nslab = rows_per_tile // KDEPTH

    def body(tab, edg, zrows, out_a, out_b, idxb, rowsb, acc, gsem,
             ssem, isem):
        c = lax.axis_index("c")
        s = lax.axis_index("s")
        pltpu.sync_copy(zrows, acc.at[pl.ds(s * stripe, stripe)])
        plsc.subcore_barrier()
        lo = (c * NTILE + s) * rows_per_tile
        _agg_loop(tab, edg, idxb, rowsb, acc, gsem, ssem, isem, lo,
                  nslab)
        plsc.subcore_barrier()

        @pl.when(c == 0)
        def _():
            pltpu.sync_copy(acc.at[pl.ds(s * stripe, stripe)],
                            out_a.at[pl.ds(s * stripe, stripe)])

        @pl.when(c == 1)
        def _():
            pltpu.sync_copy(acc.at[pl.ds(s * stripe, stripe)],
                            out_b.at[pl.ds(s * stripe, stripe)])

    return pl.kernel(
        body,
        out_type=(jax.ShapeDtypeStruct((np_, FP), jnp.float32),
                  jax.ShapeDtypeStruct((np_, FP), jnp.float32)),
        mesh=_sc_mesh(),
        compiler_params=pltpu.CompilerParams(use_tc_tiling_on_sc=False),
        scratch_types=_sc_scratch(np_),
    )


def _make_sc_agg2(np_, r_pad):
    stripe = np_ // NTILE
    rows_per_tile = r_pad // NTILE
    nslab = rows_per_tile // KDEPTH

    def body(tab_a, tab_b, edg, zrows, out_a, out_b, idxb, rowsb, acc,
             gsem, ssem, isem):
        c = lax.axis_index("c")
        s = lax.axis_index("s")
        pltpu.sync_copy(zrows, acc.at[pl.ds(s * stripe, stripe)])
        plsc.subcore_barrier()
        lo = s * rows_per_tile

        @pl.when(c == 0)
        def _():
            _agg_loop(tab_a, edg, idxb, rowsb, acc, gsem, ssem, isem,
                      lo, nslab)

        @pl.when(c == 1)
        def _():
            _agg_loop(tab_b, edg, idxb, rowsb, acc, gsem, ssem, isem,
                      lo, nslab)

        plsc.subcore_barrier()

        @pl.when(c == 0)
        def _():
            pltpu.sync_copy(acc.at[pl.ds(s * stripe, stripe)],
                            out_a.at[pl.ds(s * stripe, stripe)])

        @pl.when(c == 1)
        def _():
            pltpu.sync_copy(acc.at[pl.ds(s * stripe, stripe)],
                            out_b.at[pl.ds(s * stripe, stripe)])

    return pl.kernel(
        body,
        out_type=(jax.ShapeDtypeStruct((np_, FP), jnp.float32),
                  jax.ShapeDtypeStruct((np_, FP), jnp.float32)),
        mesh=_sc_mesh(),
        compiler_params=pltpu.CompilerParams(use_tc_tiling_on_sc=False),
        scratch_types=_sc_scratch(np_),
    )


def _tc1a_body(a0, a1, xp, wl, wr, b, t_ref, st_ref, inv_ref, ssum, ssq):
    i = pl.program_id(0)
    cnt = a0[:, FP - 1:FP] + a1[:, FP - 1:FP]
    inv = 1.0 / jnp.maximum(cnt, 1.0)
    agg = (a0[...] + a1[...]) * inv
    t = (jnp.dot(agg, wl[...], preferred_element_type=jnp.float32)
         + jnp.dot(xp[...], wr[...], preferred_element_type=jnp.float32)
         + b[...])
    t_ref[...] = t
    inv_ref[...] = inv

    @pl.when(i == 0)
    def _():
        ssum[...] = jnp.zeros_like(ssum)
        ssq[...] = jnp.zeros_like(ssq)

    ssum[...] += jnp.sum(t, axis=0, keepdims=True)
    ssq[...] += jnp.sum(t * t, axis=0, keepdims=True)
    st_ref[0:1, :] = ssum[...]
    st_ref[1:2, :] = ssq[...]


def _tc1b_body(n, t_in, st, g, be, ha, hb):
    rn = 1.0 / n
    mean = st[0:1, :] * rn
    var = st[1:2, :] * rn - mean * mean
    isd = lax.rsqrt(var + 1e-5)
    h = jnp.maximum((t_in[...] - mean) * isd * g[...] + be[...], 0.0)
    ha[...] = h[:, :FP]
    hb[...] = h[:, FP:]


def _tc2a_body(a2a, a2b, h1a, h1b, invc, wl, wr, b, t_ref, st_ref, ssum,
               ssq):
    i = pl.program_id(0)
    inv = invc[...]
    t = (jnp.dot(a2a[...] * inv, wl[0:FP, :],
                 preferred_element_type=jnp.float32)
         + jnp.dot(a2b[...] * inv, wl[FP:2 * FP, :],
                   preferred_element_type=jnp.float32)
         + jnp.dot(h1a[...], wr[0:FP, :],
                   preferred_element_type=jnp.float32)
         + jnp.dot(h1b[...], wr[FP:2 * FP, :],
                   preferred_element_type=jnp.float32)
         + b[...])
    t_ref[...] = t

    @pl.when(i == 0)
    def _():
        ssum[...] = jnp.zeros_like(ssum)
        ssq[...] = jnp.zeros_like(ssq)

    ssum[...] += jnp.sum(t, axis=0, keepdims=True)
    ssq[...] += jnp.sum(t * t, axis=0, keepdims=True)
    st_ref[0:1, :] = ssum[...]
    st_ref[1:2, :] = ssq[...]


def _tc2b_body(n, t_in, st, g, be, wm1, bm1, wm2, bm2, out):
    rn = 1.0 / n
    mean = st[0:1, :] * rn
    var = st[1:2, :] * rn - mean * mean
    isd = lax.rsqrt(var + 1e-5)
    h = jnp.maximum((t_in[...] - mean) * isd * g[...] + be[...], 0.0)
    m = jnp.maximum(
        jnp.dot(h, wm1[...], preferred_element_type=jnp.float32) + bm1[...],
        0.0)
    out[...] = (jnp.dot(m, wm2[...], preferred_element_type=jnp.float32)
                + bm2[...])


def kernel(x, edge_index, W1l, b1l, W1r, g1, be1, W2l, b2l, W2r, g2, be2,
           Wm1, bm1, Wm2, bm2):
    n, f = x.shape
    e = edge_index.shape[1]
    nb = n // BLK
    np_ = (n // 128 + 1) * 128

    src = edge_index[0].astype(jnp.int32)
    dst = edge_index[1].astype(jnp.int32)
    unit = NSC * NTILE * KDEPTH * LANES
    e_pad = ((e + unit - 1) // unit) * unit
    pad = e_pad - e
    if pad:
        src = jnp.concatenate([src, jnp.zeros((pad,), jnp.int32)])
        dst = jnp.concatenate([dst, jnp.full((pad,), n, jnp.int32)])
    edg = jnp.stack([src.reshape(-1, LANES), dst.reshape(-1, LANES)], axis=1)
    r_pad = e_pad // LANES

    xp = jnp.concatenate(
        [x, jnp.zeros((n, FP - 1 - f), x.dtype), jnp.ones((n, 1), x.dtype)],
        axis=1)
    zrows = jnp.zeros((np_ // NTILE, FP), jnp.float32)
    w1l = jnp.concatenate([W1l, jnp.zeros((FP - f, W1l.shape[1]))], axis=0)
    w1r = jnp.concatenate([W1r, jnp.zeros((FP - f, W1r.shape[1]))], axis=0)

    a1a, a1b = _make_sc_agg1(np_, r_pad)(xp, edg, zrows)

    row_spec = pl.BlockSpec((BLK, FP), lambda i: (i, 0))
    full = lambda shape: pl.BlockSpec(shape, lambda i: (0, 0))

    t1, st1, invc = pl.pallas_call(
        _tc1a_body,
        grid=(nb,),
        in_specs=[row_spec, row_spec, row_spec, full((FP, 32)),
                  full((FP, 32)), full((1, 32))],
        out_specs=[pl.BlockSpec((BLK, 32), lambda i: (i, 0)),
                   full((2, 32)),
                   pl.BlockSpec((BLK, 1), lambda i: (i, 0))],
        out_shape=[jax.ShapeDtypeStruct((n, 32), jnp.float32),
                   jax.ShapeDtypeStruct((2, 32), jnp.float32),
                   jax.ShapeDtypeStruct((n, 1), jnp.float32)],
        scratch_shapes=[pltpu.VMEM((1, 32), jnp.float32),
                        pltpu.VMEM((1, 32), jnp.float32)],
    )(a1a, a1b, xp, w1l, w1r, b1l.reshape(1, -1))

    h1a, h1b = pl.pallas_call(
        functools.partial(_tc1b_body, n),
        grid=(nb,),
        in_specs=[pl.BlockSpec((BLK, 32), lambda i: (i, 0)), full((2, 32)),
                  full((1, 32)), full((1, 32))],
        out_specs=[row_spec, row_spec],
        out_shape=[jax.ShapeDtypeStruct((n, FP), jnp.float32),
                   jax.ShapeDtypeStruct((n, FP), jnp.float32)],
    )(t1, st1, g1.reshape(1, -1), be1.reshape(1, -1))

    a2a, a2b = _make_sc_agg2(np_, r_pad)(h1a, h1b, edg, zrows)

    t2, st2 = pl.pallas_call(
        _tc2a_body,
        grid=(nb,),
        in_specs=[row_spec, row_spec, row_spec, row_spec,
                  pl.BlockSpec((BLK, 1), lambda i: (i, 0)),
                  full((32, 32)), full((32, 32)), full((1, 32))],
        out_specs=[pl.BlockSpec((BLK, 32), lambda i: (i, 0)),
                   full((2, 32))],
        out_shape=[jax.ShapeDtypeStruct((n, 32), jnp.float32),
                   jax.ShapeDtypeStruct((2, 32), jnp.float32)],
        scratch_shapes=[pltpu.VMEM((1, 32), jnp.float32),
                        pltpu.VMEM((1, 32), jnp.float32)],
    )(a2a, a2b, h1a, h1b, invc, W2l, W2r, b2l.reshape(1, -1))

    out = pl.pallas_call(
        functools.partial(_tc2b_body, n),
        grid=(nb,),
        in_specs=[pl.BlockSpec((BLK, 32), lambda i: (i, 0)), full((2, 32)),
                  full((1, 32)), full((1, 32)), full((32, 64)),
                  full((1, 64)), full((64, 1)), full((1, 1))],
        out_specs=pl.BlockSpec((BLK, 1), lambda i: (i, 0)),
        out_shape=jax.ShapeDtypeStruct((n, 1), jnp.float32),
    )(t2, st2, g2.reshape(1, -1), be2.reshape(1, -1), Wm1,
      bm1.reshape(1, -1), Wm2, bm2.reshape(1, -1))

    return out.reshape(n)


# speedup vs baseline: 41.9756x; 1.0262x over previous
import jax
import jax.numpy as jnp
from jax import lax
from jax.experimental import pallas as pl
from jax.experimental.pallas import tpu as pltpu
from jax.experimental.pallas import tpu_sc as plsc

FP = 16
LANES = 128
KDEPTH = 5
NSC = 2
NTILE = 16
BLK = 2000


def _sc_mesh():
    return plsc.VectorSubcoreMesh(
        core_axis_name="c", subcore_axis_name="s", num_cores=NSC,
        num_subcores=NTILE)


def _agg_loop(tab, edg, idxb, rowsb, acc, gsem, ssem, isem, slab_lo, nslab):
    def idx_load(t):
        p = lax.rem(t, 3)
        r0 = (slab_lo + t) * KDEPTH
        pltpu.async_copy(edg.at[0, pl.ds(r0, KDEPTH)], idxb.at[p, 0], isem)
        pltpu.async_copy(edg.at[1, pl.ds(r0, KDEPTH)], idxb.at[p, 1], isem)

    def gathers(t):
        p = lax.rem(t, 3)
        b = lax.rem(t, 2)
        for j in range(KDEPTH):
            pltpu.async_copy(tab.at[idxb.at[p, 0, j]], rowsb.at[b, j], gsem)

    def drain_scatters():
        for j in range(KDEPTH):
            pltpu.make_async_copy(rowsb.at[0, j], acc.at[idxb.at[0, 1, j]],
                                  ssem).wait()

    r_start = slab_lo * KDEPTH
    pltpu.sync_copy(edg.at[0, pl.ds(r_start, KDEPTH)], idxb.at[0, 0])
    pltpu.sync_copy(edg.at[1, pl.ds(r_start, KDEPTH)], idxb.at[0, 1])
    gathers(0)
    idx_load(1)

    def slab(t, carry):
        p = lax.rem(t, 3)
        b = lax.rem(t, 2)
        for j in range(KDEPTH):
            pltpu.make_async_copy(tab.at[idxb.at[p, 0, j]], rowsb.at[b, j],
                                  gsem).wait()
        for j in range(KDEPTH):
            pltpu.async_copy(rowsb.at[b, j], acc.at[idxb.at[p, 1, j]], ssem,
                             add=True)

        @pl.when(t + 1 < nslab)
        def _():
            pltpu.make_async_copy(edg.at[0, pl.ds(r_start, KDEPTH)],
                                  idxb.at[0, 0], isem).wait()
            pltpu.make_async_copy(edg.at[1, pl.ds(r_start, KDEPTH)],
                                  idxb.at[0, 1], isem).wait()

        @pl.when(t >= 1)
        def _():
            drain_scatters()

        @pl.when(t + 2 < nslab)
        def _():
            idx_load(t + 2)

        @pl.when(t + 1 < nslab)
        def _():
            gathers(t + 1)

        return carry

    lax.fori_loop(0, nslab, slab, 0)
    drain_scatters()


def _sc_scratch(np_):
    return [
        pltpu.VMEM((3, 2, KDEPTH, LANES), jnp.int32),
        pltpu.VMEM((2, KDEPTH, LANES, FP), jnp.float32),
        pltpu.VMEM_SHARED((np_, FP), jnp.float32),
        pltpu.SemaphoreType.DMA,
        pltpu.SemaphoreType.DMA,
        pltpu.SemaphoreType.DMA,
    ]


def _make_sc_agg1(np_, rows):
    stripe = np_ // NTILE
    slabs_per_sc = rows // (NSC * KDEPTH)

    def body(tab, edg, zrows, out_a, out_b, idxb, rowsb, acc, gsem,
             ssem, isem):
        c = lax.axis_index("c")
        s = lax.axis_index("s")
        pltpu.sync_copy(zrows, acc.at[pl.ds(s * stripe, stripe)])
        plsc.subcore_barrier()
        lo = c * slabs_per_sc + (s * slabs_per_sc) // NTILE
        hi = c * slabs_per_sc + ((s + 1) * slabs_per_sc) // NTILE
        _agg_loop(tab, edg, idxb, rowsb, acc, gsem, ssem, isem, lo, hi - lo)
        plsc.subcore_barrier()

        @pl.when(c == 0)
        def _():
            pltpu.sync_copy(acc.at[pl.ds(s * stripe, stripe)],
                            out_a.at[pl.ds(s * stripe, stripe)])

        @pl.when(c == 1)
        def _():
            pltpu.sync_copy(acc.at[pl.ds(s * stripe, stripe)],
                            out_b.at[pl.ds(s * stripe, stripe)])

    return pl.kernel(
        body,
        out_type=(jax.ShapeDtypeStruct((np_, FP), jnp.float32),
                  jax.ShapeDtypeStruct((np_, FP), jnp.float32)),
        mesh=_sc_mesh(),
        compiler_params=pltpu.CompilerParams(use_tc_tiling_on_sc=False),
        scratch_types=_sc_scratch(np_),
    )


def _make_sc_agg2(np_, rows):
    stripe = np_ // NTILE
    nslabs = rows // KDEPTH

    def body(tab_a, tab_b, edg, zrows, out_a, out_b, idxb, rowsb, acc,
             gsem, ssem, isem):
        c = lax.axis_index("c")
        s = lax.axis_index("s")
        pltpu.sync_copy(zrows, acc.at[pl.ds(s * stripe, stripe)])
        plsc.subcore_barrier()
        lo = (s * nslabs) // NTILE
        hi = ((s + 1) * nslabs) // NTILE

        @pl.when(c == 0)
        def _():
            _agg_loop(tab_a, edg, idxb, rowsb, acc, gsem, ssem, isem,
                      lo, hi - lo)

        @pl.when(c == 1)
        def _():
            _agg_loop(tab_b, edg, idxb, rowsb, acc, gsem, ssem, isem,
                      lo, hi - lo)

        plsc.subcore_barrier()

        @pl.when(c == 0)
        def _():
            pltpu.sync_copy(acc.at[pl.ds(s * stripe, stripe)],
                            out_a.at[pl.ds(s * stripe, stripe)])

        @pl.when(c == 1)
        def _():
            pltpu.sync_copy(acc.at[pl.ds(s * stripe, stripe)],
                            out_b.at[pl.ds(s * stripe, stripe)])

    return pl.kernel(
        body,
        out_type=(jax.ShapeDtypeStruct((np_, FP), jnp.float32),
                  jax.ShapeDtypeStruct((np_, FP), jnp.float32)),
        mesh=_sc_mesh(),
        compiler_params=pltpu.CompilerParams(use_tc_tiling_on_sc=False),
        scratch_types=_sc_scratch(np_),
    )


def _tc1_body(n, a0, a1, xp, wl, wr, b, g, be, ha, hb, inv_ref, ssum, ssq):
    ph = pl.program_id(0)
    i = pl.program_id(1)
    cnt = a0[:, FP - 1:FP] + a1[:, FP - 1:FP]
    inv = 1.0 / jnp.maximum(cnt, 1.0)
    agg = (a0[...] + a1[...]) * inv
    t = (jnp.dot(agg, wl[...], preferred_element_type=jnp.float32)
         + jnp.dot(xp[...], wr[...], preferred_element_type=jnp.float32)
         + b[...])

    @pl.when((ph == 0) & (i == 0))
    def _():
        ssum[...] = jnp.zeros_like(ssum)
        ssq[...] = jnp.zeros_like(ssq)

    @pl.when(ph == 0)
    def _():
        ssum[...] += jnp.sum(t, axis=0, keepdims=True)
        ssq[...] += jnp.sum(t * t, axis=0, keepdims=True)

    rn = 1.0 / n
    mean = ssum[...] * rn
    var = ssq[...] * rn - mean * mean
    isd = lax.rsqrt(var + 1e-5)
    h = jnp.maximum((t - mean) * isd * g[...] + be[...], 0.0)
    ha[...] = h[:, :FP]
    hb[...] = h[:, FP:]
    inv_ref[...] = inv


def _tc2_body(n, a2a, a2b, h1a, h1b, invc, wl, wr, b, g, be, wm1, bm1, wm2,
              bm2, out, ssum, ssq):
    ph = pl.program_id(0)
    i = pl.program_id(1)
    inv = invc[...]
    t = (jnp.dot(a2a[...] * inv, wl[0:FP, :],
                 preferred_element_type=jnp.float32)
         + jnp.dot(a2b[...] * inv, wl[FP:2 * FP, :],
                   preferred_element_type=jnp.float32)
         + jnp.dot(h1a[...], wr[0:FP, :],
                   preferred_element_type=jnp.float32)
         + jnp.dot(h1b[...], wr[FP:2 * FP, :],
                   preferred_element_type=jnp.float32)
         + b[...])

    @pl.when((ph == 0) & (i == 0))
    def _():
        ssum[...] = jnp.zeros_like(ssum)
        ssq[...] = jnp.zeros_like(ssq)

    @pl.when(ph == 0)
    def _():
        ssum[...] += jnp.sum(t, axis=0, keepdims=True)
        ssq[...] += jnp.sum(t * t, axis=0, keepdims=True)

    rn = 1.0 / n
    mean = ssum[...] * rn
    var = ssq[...] * rn - mean * mean
    isd = lax.rsqrt(var + 1e-5)
    h = jnp.maximum((t - mean) * isd * g[...] + be[...], 0.0)
    m = jnp.maximum(
        jnp.dot(h, wm1[...], preferred_element_type=jnp.float32) + bm1[...],
        0.0)
    out[...] = (jnp.dot(m, wm2[...], preferred_element_type=jnp.float32)
                + bm2[...])


def kernel(x, edge_index, W1l, b1l, W1r, g1, be1, W2l, b2l, W2r, g2, be2,
           Wm1, bm1, Wm2, bm2):
    import functools
    n, f = x.shape
    e = edge_index.shape[1]
    nb = n // BLK
    np_ = (n // 128 + 1) * 128
    rows = e // LANES

    edg = edge_index.astype(jnp.int32).reshape(2, rows, LANES)

    xp = jnp.concatenate(
        [x, jnp.zeros((n, FP - 1 - f), x.dtype), jnp.ones((n, 1), x.dtype)],
        axis=1)
    zrows = jnp.zeros((np_ // NTILE, FP), jnp.float32)
    w1l = jnp.concatenate([W1l, jnp.zeros((FP - f, W1l.shape[1]))], axis=0)
    w1r = jnp.concatenate([W1r, jnp.zeros((FP - f, W1r.shape[1]))], axis=0)

    a1a, a1b = _make_sc_agg1(np_, rows)(xp, edg, zrows)

    row_spec = pl.BlockSpec((BLK, FP), lambda ph, i: (i, 0))
    col_spec = pl.BlockSpec((BLK, 1), lambda ph, i: (i, 0))
    full = lambda shape: pl.BlockSpec(shape, lambda ph, i: (0, 0))

    h1a, h1b, invc = pl.pallas_call(
        functools.partial(_tc1_body, n),
        grid=(2, nb),
        in_specs=[row_spec, row_spec, row_spec, full((FP, 32)),
                  full((FP, 32)), full((1, 32)), full((1, 32)),
                  full((1, 32))],
        out_specs=[row_spec, row_spec, col_spec],
        out_shape=[jax.ShapeDtypeStruct((n, FP), jnp.float32),
                   jax.ShapeDtypeStruct((n, FP), jnp.float32),
                   jax.ShapeDtypeStruct((n, 1), jnp.float32)],
        scratch_shapes=[pltpu.VMEM((1, 32), jnp.float32),
                        pltpu.VMEM((1, 32), jnp.float32)],
    )(a1a, a1b, xp, w1l, w1r, b1l.reshape(1, -1), g1.reshape(1, -1),
      be1.reshape(1, -1))

    a2a, a2b = _make_sc_agg2(np_, rows)(h1a, h1b, edg, zrows)

    out = pl.pallas_call(
        functools.partial(_tc2_body, n),
        grid=(2, nb),
        in_specs=[row_spec, row_spec, row_spec, row_spec, col_spec,
                  full((32, 32)), full((32, 32)), full((1, 32)),
                  full((1, 32)), full((1, 32)), full((32, 64)),
                  full((1, 64)), full((64, 1)), full((1, 1))],
        out_specs=col_spec,
        out_shape=jax.ShapeDtypeStruct((n, 1), jnp.float32),
        scratch_shapes=[pltpu.VMEM((1, 32), jnp.float32),
                        pltpu.VMEM((1, 32), jnp.float32)],
    )(a2a, a2b, h1a, h1b, invc, W2l, W2r, b2l.reshape(1, -1),
      g2.reshape(1, -1), be2.reshape(1, -1), Wm1, bm1.reshape(1, -1), Wm2,
      bm2.reshape(1, -1))

    return out.reshape(n)


# speedup vs baseline: 45.8038x; 1.0912x over previous
import jax
import jax.numpy as jnp
from jax import lax
from jax.experimental import pallas as pl
from jax.experimental.pallas import tpu as pltpu
from jax.experimental.pallas import tpu_sc as plsc

FP = 16
LANES = 128
KDEPTH = 5
NSC = 2
NTILE = 16
WB = 256


def _sc_mesh():
    return plsc.VectorSubcoreMesh(
        core_axis_name="c", subcore_axis_name="s", num_cores=NSC,
        num_subcores=NTILE)


def _agg_loop(tab, edg, idxb, rowsb, acc, gsem, ssem, isem, slab_lo, nslab):
    def idx_load(t):
        p = lax.rem(t, 3)
        r0 = (slab_lo + t) * KDEPTH
        pltpu.async_copy(edg.at[0, pl.ds(r0, KDEPTH)], idxb.at[p, 0], isem)
        pltpu.async_copy(edg.at[1, pl.ds(r0, KDEPTH)], idxb.at[p, 1], isem)

    def gathers(t):
        p = lax.rem(t, 3)
        b = lax.rem(t, 2)
        for j in range(KDEPTH):
            pltpu.async_copy(tab.at[idxb.at[p, 0, j]], rowsb.at[b, j], gsem)

    def drain_scatters():
        for j in range(KDEPTH):
            pltpu.make_async_copy(rowsb.at[0, j], acc.at[idxb.at[0, 1, j]],
                                  ssem).wait()

    r_start = slab_lo * KDEPTH
    pltpu.sync_copy(edg.at[0, pl.ds(r_start, KDEPTH)], idxb.at[0, 0])
    pltpu.sync_copy(edg.at[1, pl.ds(r_start, KDEPTH)], idxb.at[0, 1])
    gathers(0)
    idx_load(1)

    def slab(t, carry):
        p = lax.rem(t, 3)
        b = lax.rem(t, 2)
        for j in range(KDEPTH):
            pltpu.make_async_copy(tab.at[idxb.at[p, 0, j]], rowsb.at[b, j],
                                  gsem).wait()
        for j in range(KDEPTH):
            pltpu.async_copy(rowsb.at[b, j], acc.at[idxb.at[p, 1, j]], ssem,
                             add=True)

        @pl.when(t + 1 < nslab)
        def _():
            pltpu.make_async_copy(edg.at[0, pl.ds(r_start, KDEPTH)],
                                  idxb.at[0, 0], isem).wait()
            pltpu.make_async_copy(edg.at[1, pl.ds(r_start, KDEPTH)],
                                  idxb.at[0, 1], isem).wait()

        @pl.when(t >= 1)
        def _():
            drain_scatters()

        @pl.when(t + 2 < nslab)
        def _():
            idx_load(t + 2)

        @pl.when(t + 1 < nslab)
        def _():
            gathers(t + 1)

        return carry

    lax.fori_loop(0, nslab, slab, 0)
    drain_scatters()


def _sc_scratch(np_):
    return [
        pltpu.VMEM((3, 2, KDEPTH, LANES), jnp.int32),
        pltpu.VMEM((2, KDEPTH, LANES, FP), jnp.float32),
        pltpu.VMEM_SHARED((np_, FP), jnp.float32),
        pltpu.SemaphoreType.DMA,
        pltpu.SemaphoreType.DMA,
        pltpu.SemaphoreType.DMA,
    ]


def _make_sc_agg1(np_, rows):
    stripe = np_ // NTILE
    slabs_per_sc = rows // (NSC * KDEPTH)

    def body(tab, edg, zrows, out_a, out_b, idxb, rowsb, acc, gsem,
             ssem, isem):
        c = lax.axis_index("c")
        s = lax.axis_index("s")
        pltpu.sync_copy(zrows, acc.at[pl.ds(s * stripe, stripe)])
        plsc.subcore_barrier()
        lo = c * slabs_per_sc + (s * slabs_per_sc) // NTILE
        hi = c * slabs_per_sc + ((s + 1) * slabs_per_sc) // NTILE
        _agg_loop(tab, edg, idxb, rowsb, acc, gsem, ssem, isem, lo, hi - lo)
        plsc.subcore_barrier()

        @pl.when(c == 0)
        def _():
            pltpu.sync_copy(acc.at[pl.ds(s * stripe, stripe)],
                            out_a.at[pl.ds(s * stripe, stripe)])

        @pl.when(c == 1)
        def _():
            pltpu.sync_copy(acc.at[pl.ds(s * stripe, stripe)],
                            out_b.at[pl.ds(s * stripe, stripe)])

    return pl.kernel(
        body,
        out_type=(jax.ShapeDtypeStruct((np_, FP), jnp.float32),
                  jax.ShapeDtypeStruct((np_, FP), jnp.float32)),
        mesh=_sc_mesh(),
        compiler_params=pltpu.CompilerParams(use_tc_tiling_on_sc=False),
        scratch_types=_sc_scratch(np_),
    )


def _make_sc_agg2(np_, rows):
    stripe = np_ // NTILE
    nslabs = rows // KDEPTH

    def body(tab_a, tab_b, edg, zrows, out_a, out_b, idxb, rowsb, acc,
             gsem, ssem, isem):
        c = lax.axis_index("c")
        s = lax.axis_index("s")
        pltpu.sync_copy(zrows, acc.at[pl.ds(s * stripe, stripe)])
        plsc.subcore_barrier()
        lo = (s * nslabs) // NTILE
        hi = ((s + 1) * nslabs) // NTILE

        @pl.when(c == 0)
        def _():
            _agg_loop(tab_a, edg, idxb, rowsb, acc, gsem, ssem, isem,
                      lo, hi - lo)

        @pl.when(c == 1)
        def _():
            _agg_loop(tab_b, edg, idxb, rowsb, acc, gsem, ssem, isem,
                      lo, hi - lo)

        plsc.subcore_barrier()

        @pl.when(c == 0)
        def _():
            pltpu.sync_copy(acc.at[pl.ds(s * stripe, stripe)],
                            out_a.at[pl.ds(s * stripe, stripe)])

        @pl.when(c == 1)
        def _():
            pltpu.sync_copy(acc.at[pl.ds(s * stripe, stripe)],
                            out_b.at[pl.ds(s * stripe, stripe)])

    return pl.kernel(
        body,
        out_type=(jax.ShapeDtypeStruct((np_, FP), jnp.float32),
                  jax.ShapeDtypeStruct((np_, FP), jnp.float32)),
        mesh=_sc_mesh(),
        compiler_params=pltpu.CompilerParams(use_tc_tiling_on_sc=False),
        scratch_types=_sc_scratch(np_),
    )


def _node_mask(i, n):
    row = lax.broadcasted_iota(jnp.int32, (WB, LANES), 0)
    lane = lax.broadcasted_iota(jnp.int32, (WB, LANES), 1)
    nid = (i * WB + row) * 8 + lane // FP
    return nid < n


def _tc1_body(n, aw0, aw1, xw, wla, wlb, wra, wrb, sel, fm, ba, bb, ga, gb,
              bea, beb, haw, hbw, invw_ref, ssa, sqa, ssb, sqb):
    ph = pl.program_id(0)
    i = pl.program_id(1)
    valid = _node_mask(i, n)
    a = aw0[...] + aw1[...]
    cw = jnp.dot(a, sel[...], preferred_element_type=jnp.float32,
                 precision=lax.Precision.HIGHEST)
    invw = 1.0 / jnp.maximum(cw, 1.0)
    aggw = a * invw
    xv = xw[...]
    ta = (jnp.dot(aggw, wla[...], preferred_element_type=jnp.float32,
                 precision=lax.Precision.HIGHEST)
          + jnp.dot(xv, wra[...], preferred_element_type=jnp.float32,
                 precision=lax.Precision.HIGHEST)
          + ba[...])
    tb = (jnp.dot(aggw, wlb[...], preferred_element_type=jnp.float32,
                 precision=lax.Precision.HIGHEST)
          + jnp.dot(xv, wrb[...], preferred_element_type=jnp.float32,
                 precision=lax.Precision.HIGHEST)
          + bb[...])

    @pl.when((ph == 0) & (i == 0))
    def _():
        ssa[...] = jnp.zeros_like(ssa)
        sqa[...] = jnp.zeros_like(sqa)
        ssb[...] = jnp.zeros_like(ssb)
        sqb[...] = jnp.zeros_like(sqb)

    @pl.when(ph == 0)
    def _():
        tam = jnp.where(valid, ta, 0.0)
        tbm = jnp.where(valid, tb, 0.0)
        ssa[...] += jnp.sum(tam, axis=0, keepdims=True)
        sqa[...] += jnp.sum(tam * tam, axis=0, keepdims=True)
        ssb[...] += jnp.sum(tbm, axis=0, keepdims=True)
        sqb[...] += jnp.sum(tbm * tbm, axis=0, keepdims=True)

    rn = 1.0 / n

    def fold(v):
        return jnp.dot(v, fm[...], preferred_element_type=jnp.float32,
                 precision=lax.Precision.HIGHEST)

    def bn_relu(t, ss, sq, g, be):
        mean = fold(ss[...]) * rn
        var = fold(sq[...]) * rn - mean * mean
        isd = lax.rsqrt(var + 1e-5)
        return jnp.maximum((t - mean) * isd * g[...] + be[...], 0.0)

    haw[...] = bn_relu(ta, ssa, sqa, ga, bea)
    hbw[...] = bn_relu(tb, ssb, sqb, gb, beb)
    invw_ref[...] = invw


def _tc2_body(n, a2a, a2b, h1a, h1b, invw_ref, w2laa, w2lba, w2lab, w2lbb,
              w2raa, w2rba, w2rab, w2rbb, fm, ba, bb, ga, gb, bea, beb,
              wm1a, wm1b, bm1t, wm2k, bm2t, out, ssa, sqa, ssb, sqb):
    ph = pl.program_id(0)
    i = pl.program_id(1)
    valid = _node_mask(i, n)
    inv = invw_ref[...]
    ma = a2a[...] * inv
    mb = a2b[...] * inv
    h1av = h1a[...]
    h1bv = h1b[...]
    ta = (jnp.dot(ma, w2laa[...], preferred_element_type=jnp.float32,
                 precision=lax.Precision.HIGHEST)
          + jnp.dot(mb, w2lba[...], preferred_element_type=jnp.float32,
                 precision=lax.Precision.HIGHEST)
          + jnp.dot(h1av, w2raa[...], preferred_element_type=jnp.float32,
                 precision=lax.Precision.HIGHEST)
          + jnp.dot(h1bv, w2rba[...], preferred_element_type=jnp.float32,
                 precision=lax.Precision.HIGHEST)
          + ba[...])
    tb = (jnp.dot(ma, w2lab[...], preferred_element_type=jnp.float32,
                 precision=lax.Precision.HIGHEST)
          + jnp.dot(mb, w2lbb[...], preferred_element_type=jnp.float32,
                 precision=lax.Precision.HIGHEST)
          + jnp.dot(h1av, w2rab[...], preferred_element_type=jnp.float32,
                 precision=lax.Precision.HIGHEST)
          + jnp.dot(h1bv, w2rbb[...], preferred_element_type=jnp.float32,
                 precision=lax.Precision.HIGHEST)
          + bb[...])

    @pl.when((ph == 0) & (i == 0))
    def _():
        ssa[...] = jnp.zeros_like(ssa)
        sqa[...] = jnp.zeros_like(sqa)
        ssb[...] = jnp.zeros_like(ssb)
        sqb[...] = jnp.zeros_like(sqb)

    @pl.when(ph == 0)
    def _():
        tam = jnp.where(valid, ta, 0.0)
        tbm = jnp.where(valid, tb, 0.0)
        ssa[...] += jnp.sum(tam, axis=0, keepdims=True)
        sqa[...] += jnp.sum(tam * tam, axis=0, keepdims=True)
        ssb[...] += jnp.sum(tbm, axis=0, keepdims=True)
        sqb[...] += jnp.sum(tbm * tbm, axis=0, keepdims=True)

    rn = 1.0 / n

    def fold(v):
        return jnp.dot(v, fm[...], preferred_element_type=jnp.float32,
                 precision=lax.Precision.HIGHEST)

    def bn_relu(t, ss, sq, g, be):
        mean = fold(ss[...]) * rn
        var = fold(sq[...]) * rn - mean * mean
        isd = lax.rsqrt(var + 1e-5)
        return jnp.maximum((t - mean) * isd * g[...] + be[...], 0.0)

    h2a = bn_relu(ta, ssa, sqa, ga, bea)
    h2b = bn_relu(tb, ssb, sqb, gb, beb)
    m = jnp.maximum(
        jnp.dot(h2a, wm1a[...], preferred_element_type=jnp.float32,
                 precision=lax.Precision.HIGHEST)
        + jnp.dot(h2b, wm1b[...], preferred_element_type=jnp.float32,
                 precision=lax.Precision.HIGHEST)
        + bm1t[...], 0.0)
    out[...] = (jnp.dot(m, wm2k[...], preferred_element_type=jnp.float32,
                 precision=lax.Precision.HIGHEST)
                + bm2t[...])


def kernel(x, edge_index, W1l, b1l, W1r, g1, be1, W2l, b2l, W2r, g2, be2,
           Wm1, bm1, Wm2, bm2):
    import functools
    n, f = x.shape
    e = edge_index.shape[1]
    np_ = ((n + 2047) // 2048) * 2048
    nw = np_ // 8
    nb = np_ // 2048
    rows = e // LANES

    edg = edge_index.astype(jnp.int32).reshape(2, rows, LANES)

    xp = jnp.concatenate(
        [x, jnp.zeros((n, FP - 1 - f), x.dtype), jnp.ones((n, 1), x.dtype)],
        axis=1)
    xp = jnp.concatenate([xp, jnp.zeros((np_ - n, FP), x.dtype)], axis=0)
    xw = xp.reshape(nw, LANES)
    zrows = jnp.zeros((np_ // NTILE, FP), jnp.float32)
    w1l = jnp.concatenate([W1l, jnp.zeros((FP - f, W1l.shape[1]))], axis=0)
    w1r = jnp.concatenate([W1r, jnp.zeros((FP - f, W1r.shape[1]))], axis=0)

    eye8 = jnp.eye(8, dtype=jnp.float32)
    kr = lambda w: jnp.kron(eye8, w)
    rep8 = lambda v: jnp.concatenate([v.reshape(1, -1)] * 8, axis=1)
    sel = kr(jnp.zeros((FP, FP), jnp.float32).at[FP - 1, :].set(1.0))
    fm = jnp.kron(jnp.ones((8, 8), jnp.float32), jnp.eye(FP, dtype=jnp.float32))

    a1a, a1b = _make_sc_agg1(np_, rows)(xp, edg, zrows)
    aw0 = a1a.reshape(nw, LANES)
    aw1 = a1b.reshape(nw, LANES)

    wide_in = pl.BlockSpec((WB, LANES), lambda ph, i: (i, 0))
    full = lambda shape: pl.BlockSpec(shape, lambda ph, i: (0, 0))
    wvec = full((1, LANES))
    wmat = full((LANES, LANES))

    h1aw, h1bw, invw = pl.pallas_call(
        functools.partial(_tc1_body, n),
        grid=(2, nb),
        in_specs=[wide_in, wide_in, wide_in, wmat, wmat, wmat, wmat, wmat,
                  wmat, wvec, wvec, wvec, wvec, wvec, wvec],
        out_specs=[wide_in, wide_in, wide_in],
        out_shape=[jax.ShapeDtypeStruct((nw, LANES), jnp.float32),
                   jax.ShapeDtypeStruct((nw, LANES), jnp.float32),
                   jax.ShapeDtypeStruct((nw, LANES), jnp.float32)],
        scratch_shapes=[pltpu.VMEM((1, LANES), jnp.float32)] * 4,
    )(aw0, aw1, xw, kr(w1l[:, :FP]), kr(w1l[:, FP:]), kr(w1r[:, :FP]),
      kr(w1r[:, FP:]), sel, fm, rep8(b1l[:FP]), rep8(b1l[FP:]), rep8(g1[:FP]),
      rep8(g1[FP:]), rep8(be1[:FP]), rep8(be1[FP:]))

    h1a = h1aw.reshape(np_, FP)
    h1b = h1bw.reshape(np_, FP)
    a2a, a2b = _make_sc_agg2(np_, rows)(h1a, h1b, edg, zrows)
    a2aw = a2a.reshape(nw, LANES)
    a2bw = a2b.reshape(nw, LANES)

    out = pl.pallas_call(
        functools.partial(_tc2_body, n),
        grid=(2, nb),
        in_specs=[wide_in, wide_in, wide_in, wide_in, wide_in,
                  wmat, wmat, wmat, wmat, wmat, wmat, wmat, wmat, wmat,
                  wvec, wvec, wvec, wvec, wvec, wvec,
                  full((LANES, 512)), full((LANES, 512)), full((1, 512)),
                  full((512, 8)), full((1, 8))],
        out_specs=pl.BlockSpec((WB, 8), lambda ph, i: (i, 0)),
        out_shape=jax.ShapeDtypeStruct((nw, 8), jnp.float32),
        scratch_shapes=[pltpu.VMEM((1, LANES), jnp.float32)] * 4,
    )(a2aw, a2bw, h1aw, h1bw, invw,
      kr(W2l[:FP, :FP]), kr(W2l[FP:, :FP]), kr(W2l[:FP, FP:]),
      kr(W2l[FP:, FP:]), kr(W2r[:FP, :FP]), kr(W2r[FP:, :FP]),
      kr(W2r[:FP, FP:]), kr(W2r[FP:, FP:]), fm,
      rep8(b2l[:FP]), rep8(b2l[FP:]), rep8(g2[:FP]), rep8(g2[FP:]),
      rep8(be2[:FP]), rep8(be2[FP:]),
      kr(Wm1[:FP, :]), kr(Wm1[FP:, :]), rep8(bm1),
      kr(Wm2), jnp.broadcast_to(bm2.reshape(1, 1), (1, 8)))

    return out.reshape(np_)[:n]
